# baseline (speedup 1.0000x reference)
import jax
import jax.numpy as jnp
from jax import lax
from jax.experimental import pallas as pl
from jax.experimental.pallas import tpu as pltpu
from jax.experimental.pallas import tpu_sc as plsc

N = 50000
E = 1600000
B = 64
NC = 2
NS = 16
NW = NC * NS
N2 = 50176
NPT = N2 // NS
C = 2048
KD = C // 128
NFULL = E // C
CT = E - NFULL * C
KDT = CT // 128
TPW = NFULL // NW + 1
TAB = B * 27


def _body(flat_hbm, srch2_hbm, dsth2_hbm, src1_hbm, dst1_hbm,
          shx_hbm, shy_hbm, shz_hbm, lat_hbm,
          out_hbm,
          work_hbm,
          lat_v, tabx, taby, tabz,
          bounce, stage,
          sidx, didx, srows, drows, sfull, dfull,
          sxv, syv, szv, outv,
          sem):
    cid = lax.axis_index("c")
    sid = lax.axis_index("s")
    wid = sid * NC + cid
    i16 = lax.broadcasted_iota(jnp.int32, (16,), 0)

    pltpu.sync_copy(flat_hbm.at[pl.ds(sid * (NPT * 4), NPT * 4)], bounce)

    def relayout_body(k, carry):
        base = k * 16 + i16
        v = bounce[pl.ds(k * 16, 16)]
        plsc.store_scatter(stage, [base >> 3, base & 7], v)
        return carry

    lax.fori_loop(0, NPT * 4 // 16, relayout_body, 0)
    pltpu.sync_copy(stage, work_hbm.at[pl.ds(sid * (NPT // 2), NPT // 2), :])

    pltpu.sync_copy(lat_hbm, lat_v)
    for bg in range(B // 16):
        b16 = i16 + bg * 16
        L = [[plsc.load_gather(lat_v, [b16 * 9 + 3 * i + j])
              for j in range(3)] for i in range(3)]
        for code in range(27):
            s = (code // 9 - 1, (code // 3) % 3 - 1, code % 3 - 1)
            tix = b16 * 27 + code
            for j, tab in enumerate((tabx, taby, tabz)):
                acc = jnp.zeros((16,), jnp.float32)
                for i in range(3):
                    if s[i] == 1:
                        acc = acc + L[i][j]
                    elif s[i] == -1:
                        acc = acc - L[i][j]
                plsc.store_scatter(tab, [tix], acc)

    plsc.subcore_barrier()

    def process_chunk(g, kd, c):
        off = g * C
        row = g * KD
        pltpu.sync_copy(srch2_hbm.at[pl.ds(row, kd), :], sidx.at[pl.ds(0, kd), :])
        pltpu.sync_copy(dsth2_hbm.at[pl.ds(row, kd), :], didx.at[pl.ds(0, kd), :])
        pltpu.sync_copy(src1_hbm.at[pl.ds(off, c)], sfull.at[pl.ds(0, c)])
        pltpu.sync_copy(dst1_hbm.at[pl.ds(off, c)], dfull.at[pl.ds(0, c)])
        cps = []
        for j in range(kd):
            s128 = pl.ds(j * 128, 128)
            cps.append(pltpu.async_copy(work_hbm.at[sidx.at[j]], srows.at[s128], sem))
            cps.append(pltpu.async_copy(work_hbm.at[didx.at[j]], drows.at[s128], sem))
        pltpu.sync_copy(shx_hbm.at[pl.ds(off, c)], sxv.at[pl.ds(0, c)])
        pltpu.sync_copy(shy_hbm.at[pl.ds(off, c)], syv.at[pl.ds(0, c)])
        pltpu.sync_copy(shz_hbm.at[pl.ds(off, c)], szv.at[pl.ds(0, c)])
        for cp in cps:
            cp.wait()

        def vec_body(i, carry2):
            lane = pl.ds(i * 16, 16)
            rows = i16 + i * 16
            sp4 = (sfull[lane] & 1) * 4
            dp4 = (dfull[lane] & 1) * 4
            sxe = plsc.bitcast(plsc.load_gather(srows, [rows, sp4]), jnp.float32)
            sye = plsc.bitcast(plsc.load_gather(srows, [rows, sp4 + 1]), jnp.float32)
            sze = plsc.bitcast(plsc.load_gather(srows, [rows, sp4 + 2]), jnp.float32)
            b = plsc.load_gather(srows, [rows, sp4 + 3])
            dxe = plsc.bitcast(plsc.load_gather(drows, [rows, dp4]), jnp.float32)
            dye = plsc.bitcast(plsc.load_gather(drows, [rows, dp4 + 1]), jnp.float32)
            dze = plsc.bitcast(plsc.load_gather(drows, [rows, dp4 + 2]), jnp.float32)
            code = ((sxv[lane].astype(jnp.int32) + 1) * 9
                    + (syv[lane].astype(jnp.int32) + 1) * 3
                    + (szv[lane].astype(jnp.int32) + 1))
            tix = b * 27 + code
            tix = jnp.minimum(jnp.maximum(tix, 0), TAB - 1)
            vx = dxe - sxe + plsc.load_gather(tabx, [tix])
            vy = dye - sye + plsc.load_gather(taby, [tix])
            vz = dze - sze + plsc.load_gather(tabz, [tix])
            d2e = vx * vx + vy * vy + vz * vz

            yi = 0x5F3759DF - (plsc.bitcast(d2e, jnp.int32) >> 1)
            y = plsc.bitcast(yi, jnp.float32)
            h = d2e * 0.5
            y = y * (1.5 - h * y * y)
            y = y * (1.5 - h * y * y)
            y = y * (1.5 - h * y * y)
            d = jnp.where(d2e > 0.0, d2e * y, 0.0)
            outv[lane] = d
            return carry2

        lax.fori_loop(0, c // 16, vec_body, 0)
        pltpu.sync_copy(outv.at[pl.ds(0, c)], out_hbm.at[pl.ds(off, c)])

    def slot_body(t, carry):
        g = wid + t * NW

        @pl.when(g < NFULL)
        def _():
            process_chunk(g, KD, C)

        @pl.when(g == NFULL)
        def _():
            process_chunk(g, KDT, CT)

        return carry

    lax.fori_loop(0, TPW, slot_body, 0)


@jax.jit
def kernel(position, edge_index, edge_shift, lattice, batch):
    posi = jnp.concatenate(
        [lax.bitcast_convert_type(position, jnp.int32), batch[:, None]],
        axis=1)
    flat = jnp.pad(posi, ((0, N2 - N), (0, 0))).reshape(N2 * 4)
    src1 = edge_index[0]
    dst1 = edge_index[1]
    srch2 = (src1 >> 1).reshape(E // 128, 128)
    dsth2 = (dst1 >> 1).reshape(E // 128, 128)
    shx = edge_shift[:, 0]
    shy = edge_shift[:, 1]
    shz = edge_shift[:, 2]
    latf = lattice.reshape(B * 9)

    mesh = plsc.VectorSubcoreMesh(
        core_axis_name="c", subcore_axis_name="s",
        num_cores=NC, num_subcores=NS)
    run = pl.kernel(
        _body,
        out_type=jax.ShapeDtypeStruct((E,), jnp.float32),
        mesh=mesh,
        compiler_params=pltpu.CompilerParams(
            needs_layout_passes=False, use_tc_tiling_on_sc=False),
        scratch_types=[
            pltpu.HBM((N2 // 2, 8), jnp.int32),
            pltpu.VMEM((B * 9,), jnp.float32),
            pltpu.VMEM((TAB,), jnp.float32),
            pltpu.VMEM((TAB,), jnp.float32),
            pltpu.VMEM((TAB,), jnp.float32),
            pltpu.VMEM((NPT * 4,), jnp.int32),
            pltpu.VMEM((NPT // 2, 8), jnp.int32),
            pltpu.VMEM((KD, 128), jnp.int32),
            pltpu.VMEM((KD, 128), jnp.int32),
            pltpu.VMEM((C, 8), jnp.int32),
            pltpu.VMEM((C, 8), jnp.int32),
            pltpu.VMEM((C,), jnp.int32),
            pltpu.VMEM((C,), jnp.int32),
            pltpu.VMEM((C,), jnp.float32),
            pltpu.VMEM((C,), jnp.float32),
            pltpu.VMEM((C,), jnp.float32),
            pltpu.VMEM((C,), jnp.float32),
            pltpu.SemaphoreType.DMA,
        ],
    )
    return run(flat, srch2, dsth2, src1, dst1, shx, shy, shz, latf)


# speedup vs baseline: 1.2884x; 1.2884x over previous
import jax
import jax.numpy as jnp
from jax import lax
from jax.experimental import pallas as pl
from jax.experimental.pallas import tpu as pltpu
from jax.experimental.pallas import tpu_sc as plsc

N = 50000
E = 1600000
B = 64
NC = 2
NS = 16
NW = NC * NS
N2 = 50176
NPT = N2 // NS
C = 1024
KD = C // 128
NFULL = E // C
CT = E - NFULL * C
KDT = CT // 128
TPW = NFULL // NW + 1
TAB = B * 27


def _body(flat_hbm, srch2_hbm, dsth2_hbm, src1_hbm, dst1_hbm,
          shx_hbm, shy_hbm, shz_hbm, lat_hbm,
          out_hbm,
          work_hbm,
          lat_v, tabx, taby, tabz,
          bounce, stage,
          sidx0, didx0, srows0, drows0, sfull0, dfull0,
          sxv0, syv0, szv0, outv0, sem0,
          sidx1, didx1, srows1, drows1, sfull1, dfull1,
          sxv1, syv1, szv1, outv1, sem1):
    cid = lax.axis_index("c")
    sid = lax.axis_index("s")
    wid = sid * NC + cid
    i16 = lax.broadcasted_iota(jnp.int32, (16,), 0)
    bufs = (
        (sidx0, didx0, srows0, drows0, sfull0, dfull0, sxv0, syv0, szv0,
         outv0, sem0),
        (sidx1, didx1, srows1, drows1, sfull1, dfull1, sxv1, syv1, szv1,
         outv1, sem1),
    )

    pltpu.sync_copy(flat_hbm.at[pl.ds(sid * (NPT * 4), NPT * 4)], bounce)

    def relayout_body(k, carry):
        base = k * 16 + i16
        v = bounce[pl.ds(k * 16, 16)]
        plsc.store_scatter(stage, [base >> 3, base & 7], v)
        return carry

    lax.fori_loop(0, NPT * 4 // 16, relayout_body, 0)
    pltpu.sync_copy(stage, work_hbm.at[pl.ds(sid * (NPT // 2), NPT // 2), :])

    pltpu.sync_copy(lat_hbm, lat_v)
    for bg in range(B // 16):
        b16 = i16 + bg * 16
        L = [[plsc.load_gather(lat_v, [b16 * 9 + 3 * i + j])
              for j in range(3)] for i in range(3)]
        for code in range(27):
            s = (code // 9 - 1, (code // 3) % 3 - 1, code % 3 - 1)
            tix = b16 * 27 + code
            for j, tab in enumerate((tabx, taby, tabz)):
                acc = jnp.zeros((16,), jnp.float32)
                for i in range(3):
                    if s[i] == 1:
                        acc = acc + L[i][j]
                    elif s[i] == -1:
                        acc = acc - L[i][j]
                plsc.store_scatter(tab, [tix], acc)

    plsc.subcore_barrier()

    def dma_list(g, kd, c, bi):
        sidx, didx, srows, drows, sfull, dfull, sxv, syv, szv, outv, sem = \
            bufs[bi]
        off = g * C
        trips = []
        for j in range(kd):
            s128 = pl.ds(j * 128, 128)
            trips.append((work_hbm.at[sidx.at[j]], srows.at[s128], sem))
            trips.append((work_hbm.at[didx.at[j]], drows.at[s128], sem))
        trips.append((src1_hbm.at[pl.ds(off, c)], sfull.at[pl.ds(0, c)], sem))
        trips.append((dst1_hbm.at[pl.ds(off, c)], dfull.at[pl.ds(0, c)], sem))
        trips.append((shx_hbm.at[pl.ds(off, c)], sxv.at[pl.ds(0, c)], sem))
        trips.append((shy_hbm.at[pl.ds(off, c)], syv.at[pl.ds(0, c)], sem))
        trips.append((shz_hbm.at[pl.ds(off, c)], szv.at[pl.ds(0, c)], sem))
        return trips

    def fire(g, kd, c, bi):
        sidx, didx = bufs[bi][0], bufs[bi][1]
        row = g * KD
        pltpu.sync_copy(srch2_hbm.at[pl.ds(row, kd), :],
                        sidx.at[pl.ds(0, kd), :])
        pltpu.sync_copy(dsth2_hbm.at[pl.ds(row, kd), :],
                        didx.at[pl.ds(0, kd), :])
        for src, dst, sem in dma_list(g, kd, c, bi):
            pltpu.async_copy(src, dst, sem)

    def finish(g, kd, c, bi):
        _, _, srows, drows, sfull, dfull, sxv, syv, szv, outv, _ = bufs[bi]
        off = g * C
        for src, dst, sem in dma_list(g, kd, c, bi):
            pltpu.make_async_copy(src, dst, sem).wait()

        def vec_body(i, carry2):
            lane = pl.ds(i * 16, 16)
            rows = i16 + i * 16
            sp4 = (sfull[lane] & 1) * 4
            dp4 = (dfull[lane] & 1) * 4
            sxe = plsc.bitcast(plsc.load_gather(srows, [rows, sp4]), jnp.float32)
            sye = plsc.bitcast(plsc.load_gather(srows, [rows, sp4 + 1]), jnp.float32)
            sze = plsc.bitcast(plsc.load_gather(srows, [rows, sp4 + 2]), jnp.float32)
            b = plsc.load_gather(srows, [rows, sp4 + 3])
            dxe = plsc.bitcast(plsc.load_gather(drows, [rows, dp4]), jnp.float32)
            dye = plsc.bitcast(plsc.load_gather(drows, [rows, dp4 + 1]), jnp.float32)
            dze = plsc.bitcast(plsc.load_gather(drows, [rows, dp4 + 2]), jnp.float32)
            code = ((sxv[lane].astype(jnp.int32) + 1) * 9
                    + (syv[lane].astype(jnp.int32) + 1) * 3
                    + (szv[lane].astype(jnp.int32) + 1))
            tix = b * 27 + code
            tix = jnp.minimum(jnp.maximum(tix, 0), TAB - 1)
            vx = dxe - sxe + plsc.load_gather(tabx, [tix])
            vy = dye - sye + plsc.load_gather(taby, [tix])
            vz = dze - sze + plsc.load_gather(tabz, [tix])
            d2e = vx * vx + vy * vy + vz * vz

            yi = 0x5F3759DF - (plsc.bitcast(d2e, jnp.int32) >> 1)
            y = plsc.bitcast(yi, jnp.float32)
            h = d2e * 0.5
            y = y * (1.5 - h * y * y)
            y = y * (1.5 - h * y * y)
            y = y * (1.5 - h * y * y)
            d = jnp.where(d2e > 0.0, d2e * y, 0.0)
            outv[lane] = d
            return carry2

        lax.fori_loop(0, c // 16, vec_body, 0)
        pltpu.sync_copy(outv.at[pl.ds(0, c)], out_hbm.at[pl.ds(off, c)])

    def issue_slot(t, bi):
        g = wid + t * NW

        @pl.when(g < NFULL)
        def _():
            fire(g, KD, C, bi)

        @pl.when(g == NFULL)
        def _():
            fire(g, KDT, CT, bi)

    def finish_slot(t, bi):
        g = wid + t * NW

        @pl.when(g < NFULL)
        def _():
            finish(g, KD, C, bi)

        @pl.when(g == NFULL)
        def _():
            finish(g, KDT, CT, bi)

    issue_slot(0, 0)

    def pair_body(t2, carry):
        te = t2 * 2
        issue_slot(te + 1, 1)
        finish_slot(te, 0)
        issue_slot(te + 2, 0)
        finish_slot(te + 1, 1)
        return carry

    lax.fori_loop(0, (TPW + 1) // 2, pair_body, 0)


@jax.jit
def kernel(position, edge_index, edge_shift, lattice, batch):
    posi = jnp.concatenate(
        [lax.bitcast_convert_type(position, jnp.int32), batch[:, None]],
        axis=1)
    flat = jnp.pad(posi, ((0, N2 - N), (0, 0))).reshape(N2 * 4)
    src1 = edge_index[0]
    dst1 = edge_index[1]
    srch2 = (src1 >> 1).reshape(E // 128, 128)
    dsth2 = (dst1 >> 1).reshape(E // 128, 128)
    shx = edge_shift[:, 0]
    shy = edge_shift[:, 1]
    shz = edge_shift[:, 2]
    latf = lattice.reshape(B * 9)

    buf_set = [
        pltpu.VMEM((KD, 128), jnp.int32),
        pltpu.VMEM((KD, 128), jnp.int32),
        pltpu.VMEM((C, 8), jnp.int32),
        pltpu.VMEM((C, 8), jnp.int32),
        pltpu.VMEM((C,), jnp.int32),
        pltpu.VMEM((C,), jnp.int32),
        pltpu.VMEM((C,), jnp.float32),
        pltpu.VMEM((C,), jnp.float32),
        pltpu.VMEM((C,), jnp.float32),
        pltpu.VMEM((C,), jnp.float32),
        pltpu.SemaphoreType.DMA,
    ]
    mesh = plsc.VectorSubcoreMesh(
        core_axis_name="c", subcore_axis_name="s",
        num_cores=NC, num_subcores=NS)
    run = pl.kernel(
        _body,
        out_type=jax.ShapeDtypeStruct((E,), jnp.float32),
        mesh=mesh,
        compiler_params=pltpu.CompilerParams(
            needs_layout_passes=False, use_tc_tiling_on_sc=False),
        scratch_types=[
            pltpu.HBM((N2 // 2, 8), jnp.int32),
            pltpu.VMEM((B * 9,), jnp.float32),
            pltpu.VMEM((TAB,), jnp.float32),
            pltpu.VMEM((TAB,), jnp.float32),
            pltpu.VMEM((TAB,), jnp.float32),
            pltpu.VMEM((NPT * 4,), jnp.int32),
            pltpu.VMEM((NPT // 2, 8), jnp.int32),
        ] + buf_set + buf_set,
    )
    return run(flat, srch2, dsth2, src1, dst1, shx, shy, shz, latf)
